# baseline (speedup 1.0000x reference)
import functools

import jax
import jax.numpy as jnp
from jax import lax
from jax.experimental import pallas as pl
from jax.experimental.pallas import tpu as pltpu
from jax.experimental.pallas import tpu_sc as plsc

_NC = 2
_NS = 16
_NW = _NC * _NS
_CI = 16
_CJ = 25
_L = 16


def _build(B0, B1, V2, D):
    rows_per_w = B0 // _NW
    nblk = rows_per_w // _CI
    nchunk = 2 * nblk
    mesh = plsc.VectorSubcoreMesh(core_axis_name="c", subcore_axis_name="s")

    @functools.partial(
        pl.kernel,
        mesh=mesh,
        out_type=jax.ShapeDtypeStruct((B1, 8, B0 // 128, 8, 128), jnp.float32),
        scratch_types=[
            pltpu.VMEM((_CI, 2, _CJ), jnp.int32),
            pltpu.VMEM((_CI, 2, _CJ), jnp.int32),
            pltpu.VMEM((_CI * _CJ, D), jnp.float32),
            pltpu.VMEM((_CI * _CJ, D), jnp.float32),
            pltpu.VMEM((_CJ, 8, 1, 8, _CI), jnp.float32),
            pltpu.VMEM((_CJ, 8, 1, 8, _CI), jnp.float32),
            pltpu.SemaphoreType.DMA,
            pltpu.SemaphoreType.DMA,
            pltpu.SemaphoreType.DMA,
            pltpu.SemaphoreType.DMA,
        ],
        compiler_params=pltpu.CompilerParams(
            use_tc_tiling_on_sc=False, needs_layout_passes=False
        ),
    )
    def k(idx_hbm, table_hbm, out_hbm, idx0, idx1, g0, g1, t0, t1,
          gs0, gs1, ws0, ws1):
        idx_b = (idx0, idx1)
        gbuf = (g0, g1)
        tbuf = (t0, t1)
        gs = (gs0, gs1)
        ws = (ws0, ws1)
        wid = lax.axis_index("s") * _NC + lax.axis_index("c")
        row0 = wid * rows_per_w
        lanes = lax.iota(jnp.int32, _L)
        rowsel = lanes * _CJ

        def fire(ib, b):
            i_abs = row0 + ib * _CI
            pltpu.sync_copy(idx_hbm.at[pl.ds(i_abs, _CI)], idx_b[b])
            for i in range(_CI):
                pltpu.async_copy(
                    table_hbm.at[idx_b[b].at[i, b]],
                    gbuf[b].at[pl.ds(i * _CJ, _CJ)],
                    gs[b],
                )

        def gather_wait(b):
            pltpu.make_async_copy(
                table_hbm.at[pl.ds(0, _CI * _CJ)], gbuf[b], gs[b]
            ).wait()

        def transpose(b):
            def jbody(j, carry):
                rows = rowsel + j
                for c in range(D):
                    v = plsc.load_gather(
                        gbuf[b], [rows, jnp.full((_L,), c, jnp.int32)]
                    )
                    tbuf[b][j, c // 8, 0, c % 8, :] = v
                return carry

            lax.fori_loop(0, _CJ, jbody, None)

        def out_slab(ib, b):
            i_abs = row0 + ib * _CI
            i1 = i_abs // 128
            ioff = lax.rem(i_abs, 128)
            return out_hbm.at[
                pl.ds(b * _CJ, _CJ), pl.ds(0, 8), pl.ds(i1, 1),
                pl.ds(0, 8), pl.ds(ioff, _CI),
            ]

        def writeback(ib, b):
            pltpu.async_copy(tbuf[b], out_slab(ib, b), ws[b])

        def wb_wait(b):
            pltpu.make_async_copy(tbuf[b], out_slab(0, 0), ws[b]).wait()

        def body(ib, carry):
            for b in range(2):
                g = 2 * ib + b

                @pl.when(g >= 2)
                def _():
                    wb_wait(b)

                fire(ib, b)

                @pl.when(g >= 1)
                def _():
                    ib_h = ib - (1 - b)
                    gather_wait(1 - b)
                    transpose(1 - b)
                    writeback(ib_h, 1 - b)

            return carry

        lax.fori_loop(0, nblk, body, None)
        gather_wait(1)
        transpose(1)
        writeback(nblk - 1, 1)
        for b in range(2):
            wb_wait(b)

    return k


def kernel(x, table):
    B0, B1 = x.shape
    V, D = table.shape
    x2 = (x.astype(jnp.int32) * 2).reshape(B0, 2, B1 // 2)
    tt = jnp.pad(table, ((0, 0), (0, 128 - D))).reshape(2 * V, D)
    out5 = _build(B0, B1, 2 * V, D)(x2, tt)
    return jnp.transpose(out5, (2, 4, 0, 1, 3)).reshape(B0, B1, D)


# speedup vs baseline: 1.3630x; 1.3630x over previous
import functools

import jax
import jax.numpy as jnp
from jax import lax
from jax.experimental import pallas as pl
from jax.experimental.pallas import tpu as pltpu
from jax.experimental.pallas import tpu_sc as plsc

_NC = 2
_NS = 16
_NW = _NC * _NS
_CI = 16
_CJ = 25
_L = 16


def _build(B0, B1, V2, D):
    rows_per_w = B0 // _NW
    nblk = rows_per_w // _CI
    nchunk = 2 * nblk
    mesh = plsc.VectorSubcoreMesh(core_axis_name="c", subcore_axis_name="s")

    @functools.partial(
        pl.kernel,
        mesh=mesh,
        out_type=jax.ShapeDtypeStruct((B1, 8, B0 // 128, 8, 128), jnp.float32),
        scratch_types=[
            pltpu.VMEM((_CI, 2, _CJ), jnp.int32),
            pltpu.VMEM((_CI, 2, _CJ), jnp.int32),
            pltpu.VMEM((_CI * _CJ, D), jnp.float32),
            pltpu.VMEM((_CI * _CJ, D), jnp.float32),
            pltpu.VMEM((_CJ, 8, 1, 8, _CI), jnp.float32),
            pltpu.VMEM((_CJ, 8, 1, 8, _CI), jnp.float32),
            pltpu.SemaphoreType.DMA,
            pltpu.SemaphoreType.DMA,
            pltpu.SemaphoreType.DMA,
            pltpu.SemaphoreType.DMA,
        ],
        compiler_params=pltpu.CompilerParams(
            use_tc_tiling_on_sc=False, needs_layout_passes=False
        ),
    )
    def k(idx_hbm, table_hbm, out_hbm, idx0, idx1, g0, g1, t0, t1,
          gs0, gs1, ws0, ws1):
        idx_b = (idx0, idx1)
        gbuf = (g0, g1)
        tbuf = (t0, t1)
        gs = (gs0, gs1)
        ws = (ws0, ws1)
        wid = lax.axis_index("s") * _NC + lax.axis_index("c")
        row0 = wid * rows_per_w
        lanes = lax.iota(jnp.int32, _L)
        rowsel = lanes * _CJ

        def fire(ib, b):
            i_abs = row0 + ib * _CI
            pltpu.sync_copy(idx_hbm.at[pl.ds(i_abs, _CI)], idx_b[b])
            for i in range(_CI):
                pltpu.async_copy(
                    table_hbm.at[idx_b[b].at[i, b]],
                    gbuf[b].at[pl.ds(i * _CJ, _CJ)],
                    gs[b],
                )

        def gather_wait(b):
            pltpu.make_async_copy(
                table_hbm.at[pl.ds(0, _CI * _CJ)], gbuf[b], gs[b]
            ).wait()

        def transpose(b):
            def jbody(j, carry):
                rows = rowsel + j
                for c0 in range(0, D, _L):
                    vs = [
                        plsc.load_gather(
                            gbuf[b], [rows, jnp.full((_L,), c0 + k, jnp.int32)]
                        )
                        for k in range(_L)
                    ]
                    for k in range(_L):
                        c = c0 + k
                        tbuf[b][j, c // 8, 0, c % 8, :] = vs[k]
                return carry

            lax.fori_loop(0, _CJ, jbody, None)

        def out_slab(ib, b):
            i_abs = row0 + ib * _CI
            i1 = i_abs // 128
            ioff = lax.rem(i_abs, 128)
            return out_hbm.at[
                pl.ds(b * _CJ, _CJ), pl.ds(0, 8), pl.ds(i1, 1),
                pl.ds(0, 8), pl.ds(ioff, _CI),
            ]

        def writeback(ib, b):
            pltpu.async_copy(tbuf[b], out_slab(ib, b), ws[b])

        def wb_wait(b):
            pltpu.make_async_copy(tbuf[b], out_slab(0, 0), ws[b]).wait()

        def body(ib, carry):
            for b in range(2):
                g = 2 * ib + b

                @pl.when(g >= 2)
                def _():
                    wb_wait(b)

                fire(ib, b)

                @pl.when(g >= 1)
                def _():
                    ib_h = ib - (1 - b)
                    gather_wait(1 - b)
                    transpose(1 - b)
                    writeback(ib_h, 1 - b)

            return carry

        lax.fori_loop(0, nblk, body, None)
        gather_wait(1)
        transpose(1)
        writeback(nblk - 1, 1)
        for b in range(2):
            wb_wait(b)

    return k


def kernel(x, table):
    B0, B1 = x.shape
    V, D = table.shape
    x2 = (x.astype(jnp.int32) * 2).reshape(B0, 2, B1 // 2)
    tt = jnp.pad(table, ((0, 0), (0, 128 - D))).reshape(2 * V, D)
    out5 = _build(B0, B1, 2 * V, D)(x2, tt)
    return jnp.transpose(out5, (2, 4, 0, 1, 3)).reshape(B0, B1, D)


# speedup vs baseline: 1.5977x; 1.1722x over previous
import functools

import jax
import jax.numpy as jnp
from jax import lax
from jax.experimental import pallas as pl
from jax.experimental.pallas import tpu as pltpu
from jax.experimental.pallas import tpu_sc as plsc

_NC = 2
_NS = 16
_NW = _NC * _NS
_C = 8
_NBUF = 2


def _build(B0, B1, V, D):
    rows_per_w = B0 // _NW
    nchunk = rows_per_w // _C
    mesh = plsc.VectorSubcoreMesh(core_axis_name="c", subcore_axis_name="s")

    @functools.partial(
        pl.kernel,
        mesh=mesh,
        out_type=jax.ShapeDtypeStruct((B0, B1, D), jnp.float32),
        scratch_types=[
            pltpu.VMEM((_C, B1), jnp.int32),
            pltpu.VMEM((_C, B1), jnp.int32),
            pltpu.VMEM((_C, B1, D), jnp.float32),
            pltpu.VMEM((_C, B1, D), jnp.float32),
            pltpu.SemaphoreType.DMA,
            pltpu.SemaphoreType.DMA,
            pltpu.SemaphoreType.DMA,
            pltpu.SemaphoreType.DMA,
        ],
        compiler_params=pltpu.CompilerParams(use_tc_tiling_on_sc=False),
    )
    def k(idx_hbm, table_hbm, out_hbm, idx0, idx1, rows0, rows1, g0, g1, w0, w1):
        idx_b = (idx0, idx1)
        rows_b = (rows0, rows1)
        gs = (g0, g1)
        ws = (w0, w1)
        wid = lax.axis_index("s") * _NC + lax.axis_index("c")
        cbase = wid * nchunk

        def fire(g, b):
            ibase = (cbase + g) * _C
            pltpu.sync_copy(idx_hbm.at[pl.ds(ibase, _C)], idx_b[b])
            for j in range(_C):
                pltpu.async_copy(table_hbm.at[idx_b[b].at[j]], rows_b[b].at[j], gs[b])

        def gather_wait(b):
            pltpu.make_async_copy(out_hbm.at[pl.ds(0, _C)], rows_b[b], gs[b]).wait()

        def writeback(g, b):
            pltpu.async_copy(
                rows_b[b], out_hbm.at[pl.ds((cbase + g) * _C, _C)], ws[b]
            )

        def wb_wait(b):
            pltpu.make_async_copy(rows_b[b], out_hbm.at[pl.ds(0, _C)], ws[b]).wait()

        def body(i, carry):
            for b in range(_NBUF):
                g = _NBUF * i + b

                @pl.when(g >= _NBUF)
                def _():
                    wb_wait(b)

                fire(g, b)

                @pl.when(g >= 1)
                def _():
                    gather_wait(1 - b)
                    writeback(g - 1, 1 - b)

            return carry

        lax.fori_loop(0, nchunk // _NBUF, body, None)
        gather_wait((nchunk - 1) % _NBUF)
        writeback(nchunk - 1, (nchunk - 1) % _NBUF)
        for b in range(_NBUF):
            wb_wait(b)

    return k


def kernel(x, table):
    B0, B1 = x.shape
    V, D = table.shape
    x2 = x.astype(jnp.int32) * 2
    tt = jnp.pad(table, ((0, 0), (0, 128 - D))).reshape(2 * V, D)
    return _build(B0, B1, 2 * V, D)(x2, tt)
